# baseline (speedup 1.0000x reference)
import functools

import jax
import jax.numpy as jnp
from jax import lax
from jax.experimental import pallas as pl
from jax.experimental.pallas import tpu as pltpu
from jax.experimental.pallas import tpu_sc as plsc

ROWS = 128
COLS = 100000
LANES = 16
NWORKERS = 32
RPW = ROWS // NWORKERS
SUB_V = 50
SUB_E = SUB_V * LANES
NSUB = COLS // SUB_E

_NEG = -3.0e38


def _sc_row_stats(x, y):
    mesh = plsc.VectorSubcoreMesh(core_axis_name="c", subcore_axis_name="s")

    @functools.partial(
        pl.kernel,
        out_type=jax.ShapeDtypeStruct((ROWS, LANES), jnp.float32),
        mesh=mesh,
        scratch_types=[
            pltpu.VMEM((COLS,), jnp.float32),
            pltpu.VMEM((LANES,), jnp.int32),
            pltpu.VMEM((5, LANES), jnp.float32),
            pltpu.VMEM((RPW, LANES), jnp.float32),
            pltpu.SemaphoreType.DMA,
        ],
    )
    def k(x_hbm, y_hbm, out_hbm, rowbuf, ybuf, gbuf, obuf, sem0):
        wid = lax.axis_index("s") * 2 + lax.axis_index("c")
        base = wid * RPW
        iota = lax.iota(jnp.int32, LANES)
        negv = jnp.full((LANES,), _NEG, jnp.float32)

        def process_row():
            def subchunk(i, carry):
                es = list(carry)
                off = i * SUB_E
                cms = [negv] * 4
                for u in range(SUB_V):
                    v = rowbuf[pl.ds(off + u * LANES, LANES)]
                    es[u % 8] = es[u % 8] + jnp.exp(v)
                    cms[u % 4] = jnp.maximum(cms[u % 4], v)
                cm = jnp.maximum(
                    jnp.maximum(cms[0], cms[1]), jnp.maximum(cms[2], cms[3])
                )
                for sh in (1, 2, 4, 8):
                    cm = jnp.maximum(cm, cm[iota ^ sh])
                trig = cm[0] >= gbuf[4, :][0]

                @pl.when(trig)
                def _rescan():
                    u1 = u2 = u3 = u4 = u5 = negv
                    for u in range(SUB_V):
                        v = rowbuf[pl.ds(off + u * LANES, LANES)]
                        m = jnp.minimum(u1, v)
                        u1 = jnp.maximum(u1, v)
                        v = m
                        m = jnp.minimum(u2, v)
                        u2 = jnp.maximum(u2, v)
                        v = m
                        m = jnp.minimum(u3, v)
                        u3 = jnp.maximum(u3, v)
                        v = m
                        m = jnp.minimum(u4, v)
                        u4 = jnp.maximum(u4, v)
                        v = m
                        u5 = jnp.maximum(u5, v)
                    gs = [gbuf[j, :] for j in range(5)]
                    for _ in range(5):
                        g = u1
                        for sh in (1, 2, 4, 8):
                            g = jnp.maximum(g, g[iota ^ sh])
                        cand = jnp.where(u1 == g, iota, LANES)
                        for sh in (1, 2, 4, 8):
                            cand = jnp.minimum(cand, cand[iota ^ sh])
                        pm = iota == cand
                        u1 = jnp.where(pm, u2, u1)
                        u2 = jnp.where(pm, u3, u2)
                        u3 = jnp.where(pm, u4, u3)
                        u4 = jnp.where(pm, u5, u4)
                        u5 = jnp.where(pm, _NEG, u5)
                        m = g
                        for j in range(5):
                            hi = jnp.maximum(gs[j], m)
                            m = jnp.minimum(gs[j], m)
                            gs[j] = hi
                    for j in range(5):
                        gbuf[j, :] = gs[j]

                return tuple(es)

            z = jnp.zeros((LANES,), jnp.float32)
            return lax.fori_loop(0, NSUB, subchunk, (z,) * 8)

        def row_body(r, _):
            row = base + r
            pltpu.sync_copy(x_hbm.at[row], rowbuf)
            y_off = pl.multiple_of(row - (row & (LANES - 1)), LANES)
            pltpu.sync_copy(y_hbm.at[pl.ds(y_off, LANES)], ybuf)
            yl = row & (LANES - 1)
            ysv = jnp.where(iota == yl, ybuf[...], 0)
            for sh in (1, 2, 4, 8):
                ysv = ysv + ysv[iota ^ sh]
            ys = ysv[0]
            lane = ys & (LANES - 1)
            grp = rowbuf[pl.ds(ys - lane, LANES)]
            fyv = grp[jnp.broadcast_to(lane, (LANES,))]
            for j in range(5):
                gbuf[j, :] = negv

            ss = process_row()

            t_sum = ((ss[0] + ss[1]) + (ss[2] + ss[3])) + (
                (ss[4] + ss[5]) + (ss[6] + ss[7])
            )
            for sh in (1, 2, 4, 8):
                t_sum = t_sum + t_sum[iota ^ sh]
            g5 = gbuf[4, :]
            t_top5 = (
                jnp.exp(gbuf[0, :])
                + jnp.exp(gbuf[1, :])
                + jnp.exp(gbuf[2, :])
                + jnp.exp(gbuf[3, :])
                + jnp.exp(g5)
            )
            sv = (t_sum - t_top5) / jnp.exp(fyv) - jnp.where(
                fyv >= g5, 0.0, 1.0
            )
            obuf[r, :] = sv
            return 0

        lax.fori_loop(0, RPW, row_body, 0)
        pltpu.sync_copy(obuf, out_hbm.at[pl.ds(base, RPW)])

    return k(x, y)


def _tc_finish(s):
    def body(s_ref, o_ref):
        col = s_ref[:, 0:1]
        tot = jnp.sum(jnp.log(1.0 + col), axis=0, keepdims=True)
        o_ref[...] = tot * (1.0 / ROWS)

    return pl.pallas_call(
        body,
        out_shape=jax.ShapeDtypeStruct((1, 1), jnp.float32),
    )(s)


@jax.jit
def kernel(x, y):
    s = _sc_row_stats(x, y.astype(jnp.int32))
    return _tc_finish(s)[0, 0]


# speedup vs baseline: 1.0438x; 1.0438x over previous
import functools

import jax
import jax.numpy as jnp
from jax import lax
from jax.experimental import pallas as pl
from jax.experimental.pallas import tpu as pltpu
from jax.experimental.pallas import tpu_sc as plsc

ROWS = 128
COLS = 100000
LANES = 16
NWORKERS = 32
RPW = ROWS // NWORKERS
SUB_V = 25
SUB_E = SUB_V * LANES
NSUB = COLS // SUB_E

_NEG = -3.0e38


def _sc_row_stats(x, y):
    mesh = plsc.VectorSubcoreMesh(core_axis_name="c", subcore_axis_name="s")

    @functools.partial(
        pl.kernel,
        out_type=jax.ShapeDtypeStruct((ROWS, LANES), jnp.float32),
        mesh=mesh,
        scratch_types=[
            pltpu.VMEM((COLS,), jnp.float32),
            pltpu.VMEM((LANES,), jnp.int32),
            pltpu.VMEM((5, LANES), jnp.float32),
            pltpu.VMEM((RPW, LANES), jnp.float32),
            pltpu.SemaphoreType.DMA,
        ],
    )
    def k(x_hbm, y_hbm, out_hbm, rowbuf, ybuf, gbuf, obuf, sem0):
        wid = lax.axis_index("s") * 2 + lax.axis_index("c")
        base = wid * RPW
        iota = lax.iota(jnp.int32, LANES)
        negv = jnp.full((LANES,), _NEG, jnp.float32)

        def process_row():
            def subchunk(i, carry):
                s0, s1, s2, s3 = carry
                off = i * SUB_E
                cm0 = negv
                cm1 = negv
                es = [s0, s1, s2, s3]
                for u in range(SUB_V):
                    v = rowbuf[pl.ds(off + u * LANES, LANES)]
                    es[u % 4] = es[u % 4] + jnp.exp(v)
                    if u % 2 == 0:
                        cm0 = jnp.maximum(cm0, v)
                    else:
                        cm1 = jnp.maximum(cm1, v)
                cm = jnp.maximum(cm0, cm1)
                for sh in (1, 2, 4, 8):
                    cm = jnp.maximum(cm, cm[iota ^ sh])
                trig = cm[0] >= gbuf[4, :][0]

                @pl.when(trig)
                def _rescan():
                    u1 = u2 = u3 = u4 = u5 = negv
                    for u in range(SUB_V):
                        v = rowbuf[pl.ds(off + u * LANES, LANES)]
                        m = jnp.minimum(u1, v)
                        u1 = jnp.maximum(u1, v)
                        v = m
                        m = jnp.minimum(u2, v)
                        u2 = jnp.maximum(u2, v)
                        v = m
                        m = jnp.minimum(u3, v)
                        u3 = jnp.maximum(u3, v)
                        v = m
                        m = jnp.minimum(u4, v)
                        u4 = jnp.maximum(u4, v)
                        v = m
                        u5 = jnp.maximum(u5, v)
                    gs = [gbuf[j, :] for j in range(5)]
                    for _ in range(5):
                        g = u1
                        for sh in (1, 2, 4, 8):
                            g = jnp.maximum(g, g[iota ^ sh])
                        cand = jnp.where(u1 == g, iota, LANES)
                        for sh in (1, 2, 4, 8):
                            cand = jnp.minimum(cand, cand[iota ^ sh])
                        pm = iota == cand
                        u1 = jnp.where(pm, u2, u1)
                        u2 = jnp.where(pm, u3, u2)
                        u3 = jnp.where(pm, u4, u3)
                        u4 = jnp.where(pm, u5, u4)
                        u5 = jnp.where(pm, _NEG, u5)
                        m = g
                        for j in range(5):
                            hi = jnp.maximum(gs[j], m)
                            m = jnp.minimum(gs[j], m)
                            gs[j] = hi
                    for j in range(5):
                        gbuf[j, :] = gs[j]

                return (es[0], es[1], es[2], es[3])

            z = jnp.zeros((LANES,), jnp.float32)
            return lax.fori_loop(0, NSUB, subchunk, (z, z, z, z))

        def row_body(r, _):
            row = base + r
            pltpu.sync_copy(x_hbm.at[row], rowbuf)
            y_off = pl.multiple_of(row - (row & (LANES - 1)), LANES)
            pltpu.sync_copy(y_hbm.at[pl.ds(y_off, LANES)], ybuf)
            yl = row & (LANES - 1)
            ysv = jnp.where(iota == yl, ybuf[...], 0)
            for sh in (1, 2, 4, 8):
                ysv = ysv + ysv[iota ^ sh]
            ys = ysv[0]
            lane = ys & (LANES - 1)
            grp = rowbuf[pl.ds(ys - lane, LANES)]
            fyv = grp[jnp.broadcast_to(lane, (LANES,))]
            for j in range(5):
                gbuf[j, :] = negv

            s0, s1, s2, s3 = process_row()

            t_sum = (s0 + s1) + (s2 + s3)
            for sh in (1, 2, 4, 8):
                t_sum = t_sum + t_sum[iota ^ sh]
            g5 = gbuf[4, :]
            t_top5 = (
                jnp.exp(gbuf[0, :])
                + jnp.exp(gbuf[1, :])
                + jnp.exp(gbuf[2, :])
                + jnp.exp(gbuf[3, :])
                + jnp.exp(g5)
            )
            sv = (t_sum - t_top5) / jnp.exp(fyv) - jnp.where(
                fyv >= g5, 0.0, 1.0
            )
            obuf[r, :] = sv
            return 0

        lax.fori_loop(0, RPW, row_body, 0)
        pltpu.sync_copy(obuf, out_hbm.at[pl.ds(base, RPW)])

    return k(x, y)


def _tc_finish(s):
    def body(s_ref, o_ref):
        col = s_ref[:, 0:1]
        tot = jnp.sum(jnp.log(1.0 + col), axis=0, keepdims=True)
        o_ref[...] = tot * (1.0 / ROWS)

    return pl.pallas_call(
        body,
        out_shape=jax.ShapeDtypeStruct((1, 1), jnp.float32),
    )(s)


@jax.jit
def kernel(x, y):
    s = _sc_row_stats(x, y.astype(jnp.int32))
    return _tc_finish(s)[0, 0]
